# baseline (speedup 1.0000x reference)
import jax
import jax.numpy as jnp
from jax import lax
from jax.experimental import pallas as pl
from jax.experimental.pallas import tpu as pltpu
from jax.experimental.pallas import tpu_sc as plsc

NC = 2
NS = 16
NW = NC * NS
L = 16

N_ITEMS_PAD = 1_015_808


def _popularity_kernel(nnz, n_test, train_chunks, test_chunks):
    per_tile = nnz // NW
    chunk = per_tile // train_chunks
    assert per_tile * NW == nnz and chunk * train_chunks == per_tile
    assert chunk % 8 == 0
    slc = N_ITEMS_PAD // NS
    piece = slc // 8
    assert piece % (8 * L) == 0
    t_per_tile = n_test // NS
    t_chunk = t_per_tile // test_chunks
    assert t_per_tile * NS == n_test and t_chunk * test_chunks == t_per_tile
    assert t_chunk % 8 == 0

    mesh = plsc.VectorSubcoreMesh(core_axis_name="c", subcore_axis_name="s")

    def body(items_hbm, vals_hbm, tests_hbm, g0_hbm, g1_hbm,
             score_sh, zb, ib0, ib1, vb0, vb1, idx0, idx1, gb,
             isem0, isem1, vsem0, vsem1, zsem, tsem0, tsem1):
        c = lax.axis_index("c")
        s = lax.axis_index("s")
        wid = c * NS + s
        ibs, vbs = (ib0, ib1), (vb0, vb1)
        isems, vsems = (isem0, isem1), (vsem0, vsem1)
        idxs, tsems = (idx0, idx1), (tsem0, tsem1)

        base = wid * per_tile
        tbase = s * t_per_tile

        def start_train(k, b):
            pltpu.async_copy(
                items_hbm.at[pl.ds(base + k * chunk, chunk)], ibs[b], isems[b])
            pltpu.async_copy(
                vals_hbm.at[pl.ds(base + k * chunk, chunk)], vbs[b], vsems[b])

        def wait_train(b):
            pltpu.make_async_copy(
                items_hbm.at[pl.ds(0, chunk)], ibs[b], isems[b]).wait()
            pltpu.make_async_copy(
                vals_hbm.at[pl.ds(0, chunk)], vbs[b], vsems[b]).wait()

        def start_idx(k, b):
            pltpu.async_copy(
                tests_hbm.at[pl.ds(tbase + k * t_chunk, t_chunk)],
                idxs[b], tsems[b])

        def wait_idx(b):
            pltpu.make_async_copy(
                tests_hbm.at[pl.ds(0, t_chunk)], idxs[b], tsems[b]).wait()

        def zero_body(i, carry):
            for u in range(8):
                zb[pl.ds((i * 8 + u) * L, L)] = jnp.zeros((L,), jnp.float32)
            return carry

        start_train(0, 0)
        start_train(1, 1)
        start_idx(0, 0)
        lax.fori_loop(0, piece // (8 * L), zero_body, 0)
        fills = [
            pltpu.async_copy(
                zb, score_sh.at[pl.ds(s * slc + i * piece, piece)], zsem)
            for i in range(8)
        ]
        for f in fills:
            f.wait()
        plsc.subcore_barrier()

        for k in range(train_chunks):
            b = k % 2
            wait_train(b)
            pltpu.sync_copy(vbs[b], score_sh.at[ibs[b]], add=True)
            if k + 2 < train_chunks:
                start_train(k + 2, b)

        plsc.subcore_barrier()

        for k in range(test_chunks):
            b = k % 2
            if k + 1 < test_chunks:
                start_idx(k + 1, 1 - b)
            wait_idx(b)
            pltpu.sync_copy(score_sh.at[idxs[b]], gb)
            off = tbase + k * t_chunk

            @pl.when(c == 0)
            def _():
                pltpu.sync_copy(gb, g0_hbm.at[pl.ds(off, t_chunk)])

            @pl.when(c == 1)
            def _():
                pltpu.sync_copy(gb, g1_hbm.at[pl.ds(off, t_chunk)])

    return pl.kernel(
        body,
        out_type=(jax.ShapeDtypeStruct((n_test,), jnp.float32),
                  jax.ShapeDtypeStruct((n_test,), jnp.float32)),
        mesh=mesh,
        scratch_types=[
            pltpu.VMEM_SHARED((N_ITEMS_PAD,), jnp.float32),
            pltpu.VMEM((piece,), jnp.float32),
            pltpu.VMEM((chunk,), jnp.int32),
            pltpu.VMEM((chunk,), jnp.int32),
            pltpu.VMEM((chunk,), jnp.float32),
            pltpu.VMEM((chunk,), jnp.float32),
            pltpu.VMEM((t_chunk,), jnp.int32),
            pltpu.VMEM((t_chunk,), jnp.int32),
            pltpu.VMEM((t_chunk,), jnp.float32),
            pltpu.SemaphoreType.DMA,
            pltpu.SemaphoreType.DMA,
            pltpu.SemaphoreType.DMA,
            pltpu.SemaphoreType.DMA,
            pltpu.SemaphoreType.DMA,
            pltpu.SemaphoreType.DMA,
            pltpu.SemaphoreType.DMA,
        ],
    )


def _tc_combine_body(g0_ref, g1_ref, out_ref):
    out_ref[...] = (g0_ref[...] + g1_ref[...]).reshape(out_ref.shape)


def _combine_kernel_tc(n_users, n_t):
    rows = 8
    grid = (n_t + rows - 1) // rows
    return pl.pallas_call(
        _tc_combine_body,
        grid=(grid,),
        in_specs=[pl.BlockSpec((rows * n_users,), lambda t: (t,)),
                  pl.BlockSpec((rows * n_users,), lambda t: (t,))],
        out_specs=pl.BlockSpec((rows, n_users), lambda t: (t, 0)),
        out_shape=jax.ShapeDtypeStruct((n_t, n_users), jnp.float32),
    )


def kernel(train_items, train_values, test_items):
    nnz = train_items.shape[0]
    n_users, n_test_per_user = test_items.shape
    n_test = n_users * n_test_per_user

    items = train_items.astype(jnp.int32)
    tests = test_items.T.reshape(-1).astype(jnp.int32)

    g0, g1 = _popularity_kernel(nnz, n_test, train_chunks=16, test_chunks=8)(
        items, train_values, tests)
    out = _combine_kernel_tc(n_users, n_test_per_user)(g0, g1)
    return out.T


# speedup vs baseline: 1.0266x; 1.0266x over previous
import jax
import jax.numpy as jnp
from jax import lax
from jax.experimental import pallas as pl
from jax.experimental.pallas import tpu as pltpu
from jax.experimental.pallas import tpu_sc as plsc

NC = 2
NS = 16
NW = NC * NS
L = 16

N_ITEMS_PAD = 1_015_808


def _popularity_kernel(nnz, n_test, train_chunks, test_chunks):
    per_tile = nnz // NW
    chunk = per_tile // train_chunks
    assert per_tile * NW == nnz and chunk * train_chunks == per_tile
    assert chunk % 8 == 0
    slc = N_ITEMS_PAD // NS
    piece = slc // 8
    assert piece % (8 * L) == 0
    t_per_tile = n_test // NS
    t_chunk = t_per_tile // test_chunks
    assert t_per_tile * NS == n_test and t_chunk * test_chunks == t_per_tile
    assert t_chunk % 8 == 0

    mesh = plsc.VectorSubcoreMesh(core_axis_name="c", subcore_axis_name="s")

    def body(items_hbm, vals_hbm, tests_hbm, g0_hbm, g1_hbm,
             score_sh, zb, ib0, ib1, vb0, vb1, idx0, idx1, gb,
             isem0, isem1, vsem0, vsem1, zsem, tsem0, tsem1):
        c = lax.axis_index("c")
        s = lax.axis_index("s")
        wid = c * NS + s
        ibs, vbs = (ib0, ib1), (vb0, vb1)
        isems, vsems = (isem0, isem1), (vsem0, vsem1)
        idxs, tsems = (idx0, idx1), (tsem0, tsem1)

        base = wid * per_tile
        tbase = s * t_per_tile

        def start_train(k, b):
            pltpu.async_copy(
                items_hbm.at[pl.ds(base + k * chunk, chunk)], ibs[b], isems[b])
            pltpu.async_copy(
                vals_hbm.at[pl.ds(base + k * chunk, chunk)], vbs[b], vsems[b])

        def wait_train(b):
            pltpu.make_async_copy(
                items_hbm.at[pl.ds(0, chunk)], ibs[b], isems[b]).wait()
            pltpu.make_async_copy(
                vals_hbm.at[pl.ds(0, chunk)], vbs[b], vsems[b]).wait()

        def start_idx(k, b):
            pltpu.async_copy(
                tests_hbm.at[pl.ds(tbase + k * t_chunk, t_chunk)],
                idxs[b], tsems[b])

        def wait_idx(b):
            pltpu.make_async_copy(
                tests_hbm.at[pl.ds(0, t_chunk)], idxs[b], tsems[b]).wait()

        def zero_body(i, carry):
            for u in range(8):
                zb[pl.ds((i * 8 + u) * L, L)] = jnp.zeros((L,), jnp.float32)
            return carry

        start_train(0, 0)
        start_train(1, 1)
        start_idx(0, 0)
        lax.fori_loop(0, piece // (8 * L), zero_body, 0)
        fills = [
            pltpu.async_copy(
                zb, score_sh.at[pl.ds(s * slc + i * piece, piece)], zsem)
            for i in range(8)
        ]
        for f in fills:
            f.wait()
        plsc.subcore_barrier()

        def train_pair(k2, carry):
            for b in (0, 1):
                k = k2 * 2 + b
                wait_train(b)
                pltpu.sync_copy(vbs[b], score_sh.at[ibs[b]], add=True)

                @pl.when(k + 2 < train_chunks)
                def _():
                    start_train(k + 2, b)

            return carry

        lax.fori_loop(0, train_chunks // 2, train_pair, 0)
        plsc.subcore_barrier()

        def gather_pair(k2, carry):
            for b in (0, 1):
                k = k2 * 2 + b

                @pl.when(k + 1 < test_chunks)
                def _():
                    start_idx(k + 1, 1 - b)

                wait_idx(b)
                pltpu.sync_copy(score_sh.at[idxs[b]], gb)
                off = tbase + k * t_chunk

                @pl.when(c == 0)
                def _():
                    pltpu.sync_copy(gb, g0_hbm.at[pl.ds(off, t_chunk)])

                @pl.when(c == 1)
                def _():
                    pltpu.sync_copy(gb, g1_hbm.at[pl.ds(off, t_chunk)])

            return carry

        lax.fori_loop(0, test_chunks // 2, gather_pair, 0)

    return pl.kernel(
        body,
        out_type=(jax.ShapeDtypeStruct((n_test,), jnp.float32),
                  jax.ShapeDtypeStruct((n_test,), jnp.float32)),
        mesh=mesh,
        scratch_types=[
            pltpu.VMEM_SHARED((N_ITEMS_PAD,), jnp.float32),
            pltpu.VMEM((piece,), jnp.float32),
            pltpu.VMEM((chunk,), jnp.int32),
            pltpu.VMEM((chunk,), jnp.int32),
            pltpu.VMEM((chunk,), jnp.float32),
            pltpu.VMEM((chunk,), jnp.float32),
            pltpu.VMEM((t_chunk,), jnp.int32),
            pltpu.VMEM((t_chunk,), jnp.int32),
            pltpu.VMEM((t_chunk,), jnp.float32),
            pltpu.SemaphoreType.DMA,
            pltpu.SemaphoreType.DMA,
            pltpu.SemaphoreType.DMA,
            pltpu.SemaphoreType.DMA,
            pltpu.SemaphoreType.DMA,
            pltpu.SemaphoreType.DMA,
            pltpu.SemaphoreType.DMA,
        ],
    )


def _combine_kernel_tc(n_users, n_t):
    n_test = n_users * n_t

    def body(g0_hbm, g1_hbm, out_ref, b0, b1, sem0, sem1):
        cp0 = pltpu.make_async_copy(g0_hbm, b0, sem0)
        cp1 = pltpu.make_async_copy(g1_hbm, b1, sem1)
        cp0.start()
        cp1.start()
        cp0.wait()
        cp1.wait()
        out_ref[...] = (b0[...] + b1[...]).reshape(n_t, n_users)

    return pl.pallas_call(
        body,
        in_specs=[pl.BlockSpec(memory_space=pl.ANY),
                  pl.BlockSpec(memory_space=pl.ANY)],
        out_specs=pl.BlockSpec((n_t, n_users), lambda: (0, 0)),
        out_shape=jax.ShapeDtypeStruct((n_t, n_users), jnp.float32),
        scratch_shapes=[
            pltpu.VMEM((n_test,), jnp.float32),
            pltpu.VMEM((n_test,), jnp.float32),
            pltpu.SemaphoreType.DMA,
            pltpu.SemaphoreType.DMA,
        ],
    )


def kernel(train_items, train_values, test_items):
    nnz = train_items.shape[0]
    n_users, n_test_per_user = test_items.shape
    n_test = n_users * n_test_per_user

    items = train_items.astype(jnp.int32)
    tests = test_items.T.reshape(-1).astype(jnp.int32)

    g0, g1 = _popularity_kernel(nnz, n_test, train_chunks=16, test_chunks=8)(
        items, train_values, tests)
    out = _combine_kernel_tc(n_users, n_test_per_user)(g0, g1)
    return out.T
